# baseline (speedup 1.0000x reference)
import functools

import jax
import jax.numpy as jnp
from jax import lax
from jax.experimental import pallas as pl
from jax.experimental.pallas import tpu as pltpu
from jax.experimental.pallas import tpu_sc as plsc

B = 4096
NSAMP = 128
R = B * NSAMP
X = 4096
D = 64
L = 16

NC = 2
NSC = 16
NW = NC * NSC
BW = B // NW
NB = 2
CH = NB * NSAMP
NCHUNK = BW // NB
NS = 4

_ROT_A = (13, 15, 26, 6)
_ROT_B = (17, 29, 16, 24)
_PARITY = 0x1BD11BDA


def _rotl(x, r):
    return (x << r) | lax.shift_right_logical(x, 32 - r)


def _threefry_index(k0, k1, ks2, x1init):
    ks = (k0, k1, ks2)
    x0 = k0
    x1 = x1init + k1
    for g in range(5):
        rots = _ROT_A if g % 2 == 0 else _ROT_B
        for r in rots:
            x0 = x0 + x1
            x1 = _rotl(x1, r)
            x1 = x1 ^ x0
        x0 = x0 + ks[(g + 1) % 3]
        x1 = x1 + ks[(g + 2) % 3] + (g + 1)
    return (x0 ^ x1) & (X - 1)


_mesh = plsc.VectorSubcoreMesh(core_axis_name="c", subcore_axis_name="s")


@functools.partial(
    pl.kernel,
    out_type=[
        jax.ShapeDtypeStruct((R,), jnp.int32),
        jax.ShapeDtypeStruct((B, D, NSAMP), jnp.float32),
    ],
    mesh=_mesh,
    compiler_params=pltpu.CompilerParams(needs_layout_passes=False,
                                         use_tc_tiling_on_sc=False),
    scratch_types=[
        pltpu.VMEM((2, L), jnp.int32),
        pltpu.VMEM((X,), jnp.int32),
        pltpu.VMEM((NS, NB, NSAMP), jnp.int32),
        pltpu.VMEM((NS, CH), jnp.int32),
        pltpu.VMEM((NS, NB, NSAMP, D), jnp.float32),
        pltpu.VMEM((2, NB, D, NSAMP), jnp.float32),
        pltpu.SemaphoreType.DMA,
        pltpu.SemaphoreType.DMA,
        pltpu.SemaphoreType.DMA,
        pltpu.SemaphoreType.DMA,
        pltpu.SemaphoreType.DMA,
        pltpu.SemaphoreType.DMA,
        pltpu.SemaphoreType.DMA,
        pltpu.SemaphoreType.DMA,
        pltpu.SemaphoreType.DMA,
        pltpu.SemaphoreType.DMA,
    ],
)
def _sampler(keys_hbm, ids_hbm, emb_hbm, ids_out, emb_out,
             keys_v, tab_v, idx_v, oid_v, rows_v, trans_v,
             gsem0, gsem1, gsem2, gsem3, osem0, osem1,
             isem0, isem1, isem2, isem3):
    wid = lax.axis_index("s") * NC + lax.axis_index("c")
    b_base = wid * BW
    pltpu.sync_copy(keys_hbm, keys_v)
    pltpu.sync_copy(ids_hbm, tab_v)
    k0 = keys_v[0, :]
    k1 = keys_v[1, :]
    ks2 = k0 ^ k1 ^ _PARITY
    lane = lax.iota(jnp.int32, L)
    gsems = (gsem0, gsem1, gsem2, gsem3)
    osems = (osem0, osem1)
    isems = (isem0, isem1, isem2, isem3)
    zero = jnp.full((L,), 0, jnp.int32)

    def compute_idx(c, s):
        row0 = (b_base + c * NB) * NSAMP
        for q in range(NB):
            def vreg_body(j, cc, q=q):
                x1init = lane + (row0 + q * NSAMP + j * L)
                idx = _threefry_index(k0, k1, ks2, x1init)
                idx_v[s, q, pl.ds(j * L, L)] = idx
                oid_v[s, pl.ds(q * NSAMP + j * L, L)] = plsc.load_gather(
                    tab_v, [idx])
                return cc
            lax.fori_loop(0, NSAMP // L, vreg_body, 0)

    def fire_ids(c, s):
        row0 = (b_base + c * NB) * NSAMP
        return pltpu.async_copy(oid_v.at[s], ids_out.at[pl.ds(row0, CH)],
                                isems[s])

    def wait_ids(c, s):
        row0 = (b_base + c * NB) * NSAMP
        pltpu.make_async_copy(oid_v.at[s], ids_out.at[pl.ds(row0, CH)],
                              isems[s]).wait()

    def fire_gather(s):
        return [
            pltpu.async_copy(emb_hbm.at[idx_v.at[s, q]], rows_v.at[s, q],
                             gsems[s])
            for q in range(NB)
        ]

    def wait_gather(s):
        for q in range(NB):
            pltpu.make_async_copy(emb_hbm.at[idx_v.at[s, q]],
                                  rows_v.at[s, q], gsems[s]).wait()

    perm = [(lane + k) & (L - 1) for k in range(L)]
    rconst = [lane * D + pk for pk in perm]
    wconst = [pk * NSAMP + lane for pk in perm]

    def transpose(s, par):
        def t_body(t, cc):
            for q in range(NB):
                rref = rows_v.at[s, q]
                tref = trans_v.at[par, q]
                for d0 in range(0, D, L):
                    roff = t * (L * D) + d0
                    woff = d0 * NSAMP + t * L
                    vs = [
                        plsc.load_gather(rref, [zero, rconst[k] + roff])
                        for k in range(L)
                    ]
                    for k in range(L):
                        plsc.store_scatter(
                            tref, [zero, wconst[k] + woff], vs[k])
            return cc
        lax.fori_loop(0, NSAMP // L, t_body, 0)

    def fire_out(c, par):
        b0 = b_base + c * NB
        return pltpu.async_copy(trans_v.at[par], emb_out.at[pl.ds(b0, NB)],
                                osems[par])

    def wait_out(c, par):
        b0 = b_base + c * NB
        pltpu.make_async_copy(trans_v.at[par], emb_out.at[pl.ds(b0, NB)],
                              osems[par]).wait()

    for c0 in range(NS):
        compute_idx(c0, c0)
        fire_ids(c0, c0)
        fire_gather(c0)

    def body(gg, carry):
        for p in range(NS):
            g = NS * gg + p
            par = p & 1
            wait_gather(p)
            if p >= 2:
                wait_out(g - 2, par)
            else:
                @pl.when(gg >= 1)
                def _():
                    wait_out(g - 2, par)
            transpose(p, par)
            fire_out(g, par)

            @pl.when(gg + 1 < NCHUNK // NS)
            def _():
                wait_ids(g, p)
                compute_idx(g + NS, p)
                fire_ids(g + NS, p)
                fire_gather(p)
        return carry

    lax.fori_loop(0, NCHUNK // NS, body, 0)
    wait_out(NCHUNK - 2, 0)
    wait_out(NCHUNK - 1, 1)
    for s in range(NS):
        wait_ids(NCHUNK - NS + s, s)


def kernel(positive_ids, num_to_sample, sampled_candidate_ids,
           sampled_candidate_embeddings):
    del positive_ids, num_to_sample
    kd = jax.random.key_data(jax.random.split(jax.random.key(42))[1])
    keys = lax.bitcast_convert_type(kd, jnp.int32)
    keys2d = jnp.broadcast_to(keys[:, None], (2, L))
    ids_flat, emb_bdn = _sampler(
        keys2d, sampled_candidate_ids, sampled_candidate_embeddings)
    return (ids_flat.reshape(B, NSAMP),
            jnp.transpose(emb_bdn, (0, 2, 1)))


# speedup vs baseline: 1.0660x; 1.0660x over previous
import functools

import jax
import jax.numpy as jnp
from jax import lax
from jax.experimental import pallas as pl
from jax.experimental.pallas import tpu as pltpu
from jax.experimental.pallas import tpu_sc as plsc

B = 4096
NSAMP = 128
R = B * NSAMP
X = 4096
D = 64
L = 16

NC = 2
NSC = 16
NW = NC * NSC
BW = B // NW
NB = 2
CH = NB * NSAMP
NCHUNK = BW // NB
NS = 2

_ROT_A = (13, 15, 26, 6)
_ROT_B = (17, 29, 16, 24)
_PARITY = 0x1BD11BDA


def _rotl(x, r):
    return (x << r) | lax.shift_right_logical(x, 32 - r)


def _threefry_index(k0, k1, ks2, x1init):
    ks = (k0, k1, ks2)
    x0 = k0
    x1 = x1init + k1
    for g in range(5):
        rots = _ROT_A if g % 2 == 0 else _ROT_B
        for r in rots:
            x0 = x0 + x1
            x1 = _rotl(x1, r)
            x1 = x1 ^ x0
        x0 = x0 + ks[(g + 1) % 3]
        x1 = x1 + ks[(g + 2) % 3] + (g + 1)
    return (x0 ^ x1) & (X - 1)


_mesh = plsc.VectorSubcoreMesh(core_axis_name="c", subcore_axis_name="s")


@functools.partial(
    pl.kernel,
    out_type=[
        jax.ShapeDtypeStruct((R,), jnp.int32),
        jax.ShapeDtypeStruct((B, D, NSAMP), jnp.float32),
    ],
    mesh=_mesh,
    compiler_params=pltpu.CompilerParams(needs_layout_passes=False,
                                         use_tc_tiling_on_sc=False),
    scratch_types=[
        pltpu.VMEM((2, L), jnp.int32),
        pltpu.VMEM((X,), jnp.int32),
        pltpu.VMEM((NS, NB, NSAMP), jnp.int32),
        pltpu.VMEM((BW * NSAMP,), jnp.int32),
        pltpu.VMEM((NS, NB, NSAMP, D), jnp.float32),
        pltpu.VMEM((2, NB, D, NSAMP), jnp.float32),
        pltpu.SemaphoreType.DMA,
        pltpu.SemaphoreType.DMA,
        pltpu.SemaphoreType.DMA,
        pltpu.SemaphoreType.DMA,
    ],
)
def _sampler(keys_hbm, ids_hbm, emb_hbm, ids_out, emb_out,
             keys_v, tab_v, idx_v, oid_v, rows_v, trans_v,
             gsem0, gsem1, osem0, osem1):
    wid = lax.axis_index("s") * NC + lax.axis_index("c")
    b_base = wid * BW
    pltpu.sync_copy(keys_hbm, keys_v)
    pltpu.sync_copy(ids_hbm, tab_v)
    k0 = keys_v[0, :]
    k1 = keys_v[1, :]
    ks2 = k0 ^ k1 ^ _PARITY
    lane = lax.iota(jnp.int32, L)
    gsems = (gsem0, gsem1)
    osems = (osem0, osem1)
    zero = jnp.full((L,), 0, jnp.int32)

    def compute_idx(c, s):
        row0 = (b_base + c * NB) * NSAMP
        loc0 = c * CH
        for q in range(NB):
            def vreg_body(j, cc, q=q):
                x1init = lane + (row0 + q * NSAMP + j * L)
                idx = _threefry_index(k0, k1, ks2, x1init)
                idx_v[s, q, pl.ds(j * L, L)] = idx
                oid_v[pl.ds(loc0 + q * NSAMP + j * L, L)] = plsc.load_gather(
                    tab_v, [idx])
                return cc
            lax.fori_loop(0, NSAMP // L, vreg_body, 0)

    def fire_gather(s):
        return [
            pltpu.async_copy(emb_hbm.at[idx_v.at[s, q]], rows_v.at[s, q],
                             gsems[s])
            for q in range(NB)
        ]

    def wait_gather(s):
        for q in range(NB):
            pltpu.make_async_copy(emb_hbm.at[idx_v.at[s, q]],
                                  rows_v.at[s, q], gsems[s]).wait()

    perm = [(lane + k) & (L - 1) for k in range(L)]
    rconst = [lane * D + pk for pk in perm]
    wconst = [pk * NSAMP + lane for pk in perm]

    def transpose(s, par):
        def t_body(t, cc):
            for q in range(NB):
                rref = rows_v.at[s, q]
                tref = trans_v.at[par, q]
                for d0 in range(0, D, L):
                    roff = t * (L * D) + d0
                    woff = d0 * NSAMP + t * L
                    vs = [
                        plsc.load_gather(rref, [zero, rconst[k] + roff])
                        for k in range(L)
                    ]
                    for k in range(L):
                        plsc.store_scatter(
                            tref, [zero, wconst[k] + woff], vs[k])
            return cc
        lax.fori_loop(0, NSAMP // L, t_body, 0)

    def fire_out(c, par):
        b0 = b_base + c * NB
        return pltpu.async_copy(trans_v.at[par], emb_out.at[pl.ds(b0, NB)],
                                osems[par])

    def wait_out(c, par):
        b0 = b_base + c * NB
        pltpu.make_async_copy(trans_v.at[par], emb_out.at[pl.ds(b0, NB)],
                              osems[par]).wait()

    for c0 in range(NS):
        compute_idx(c0, c0)
        fire_gather(c0)

    def body(gg, carry):
        for p in range(NS):
            g = NS * gg + p
            par = p & 1
            wait_gather(p)
            if p >= 2:
                wait_out(g - 2, par)
            else:
                @pl.when(gg >= 1)
                def _():
                    wait_out(g - 2, par)
            transpose(p, par)
            fire_out(g, par)

            @pl.when(gg + 1 < NCHUNK // NS)
            def _():
                compute_idx(g + NS, p)
                fire_gather(p)
        return carry

    lax.fori_loop(0, NCHUNK // NS, body, 0)
    wait_out(NCHUNK - 2, 0)
    wait_out(NCHUNK - 1, 1)
    pltpu.sync_copy(oid_v, ids_out.at[pl.ds(b_base * NSAMP, BW * NSAMP)])


def kernel(positive_ids, num_to_sample, sampled_candidate_ids,
           sampled_candidate_embeddings):
    del positive_ids, num_to_sample
    kd = jax.random.key_data(jax.random.split(jax.random.key(42))[1])
    keys = lax.bitcast_convert_type(kd, jnp.int32)
    keys2d = jnp.broadcast_to(keys[:, None], (2, L))
    ids_flat, emb_bdn = _sampler(
        keys2d, sampled_candidate_ids, sampled_candidate_embeddings)
    return (ids_flat.reshape(B, NSAMP),
            jnp.transpose(emb_bdn, (0, 2, 1)))
